# baseline (speedup 1.0000x reference)
import jax
import jax.numpy as jnp
from jax.experimental import pallas as pl
from jax.experimental.pallas import tpu as pltpu
from jax.experimental.pallas import tpu_sc as plsc

N = 4094
NPAD = 4096
NTIPS = 2048
HID = 128

ROW_BLK = 1024
NUM_WORKERS = 32
WIN_NODES = 32
WIN_IDX = WIN_NODES * 3
NUM_WINDOWS = NPAD // WIN_NODES
WPW = NUM_WINDOWS // NUM_WORKERS
NGRP = 8


def _mm_body(x_ref, w_ref, o_ref):
    i = pl.program_id(0)
    y = jnp.dot(x_ref[...].astype(jnp.bfloat16),
                w_ref[...].astype(jnp.bfloat16),
                preferred_element_type=jnp.float32)
    row = jax.lax.broadcasted_iota(jnp.int32, (ROW_BLK, HID), 0) + i * ROW_BLK
    o_ref[...] = jnp.where(row < N, y, 0.0)


def _tc_matmul(x, w):
    return pl.pallas_call(
        _mm_body,
        grid=(NPAD // ROW_BLK,),
        in_specs=[
            pl.BlockSpec((ROW_BLK, NTIPS), lambda i: (i, 0)),
            pl.BlockSpec((NTIPS, HID), lambda i: (0, 0)),
        ],
        out_specs=pl.BlockSpec((ROW_BLK, HID), lambda i: (i, 0)),
        out_shape=jax.ShapeDtypeStruct((NPAD, HID), jnp.float32),
        compiler_params=pltpu.CompilerParams(
            dimension_semantics=("parallel",)),
    )(x, w)


def _sc_gather_sum(y_pad, e_flat):
    @pl.kernel(
        out_type=jax.ShapeDtypeStruct((NPAD, HID), jnp.float32),
        mesh=plsc.VectorSubcoreMesh(core_axis_name="c", subcore_axis_name="s"),
        scratch_types=[
            pltpu.VMEM((WPW, WIN_IDX), jnp.int32),
            pltpu.VMEM((WIN_IDX, HID), jnp.float32),
            pltpu.VMEM((WIN_IDX, HID), jnp.float32),
            pltpu.VMEM((WIN_NODES, HID), jnp.float32),
            pltpu.VMEM((WIN_NODES, HID), jnp.float32),
            pltpu.SemaphoreType.DMA,
            pltpu.SemaphoreType.DMA,
            pltpu.SemaphoreType.DMA,
            pltpu.SemaphoreType.DMA,
            pltpu.SemaphoreType.DMA,
        ],
    )
    def sc_kernel(y_hbm, idx_hbm, out_hbm,
                  idx_vmem, gat0, gat1, a0, a1,
                  semi, semg0, semg1, sema0, sema1):
        cid = jax.lax.axis_index("c")
        sid = jax.lax.axis_index("s")
        w = cid * 16 + sid

        pltpu.async_copy(idx_hbm.at[pl.ds(w * WPW, WPW)],
                         idx_vmem, semi).wait()

        bufs = ((gat0, a0, semg0, sema0), (gat1, a1, semg1, sema1))

        def start_gather(t):
            gat, _, semg, _ = bufs[t % 2]
            cg = pltpu.make_async_copy(y_hbm.at[idx_vmem.at[t]], gat, semg)
            cg.start()
            return cg

        cgs = [None] * WPW
        cas = [None] * WPW
        cgs[0] = start_gather(0)
        for t in range(WPW):
            if t + 1 < WPW:
                cgs[t + 1] = start_gather(t + 1)
            gat, abuf, _, sema = bufs[t % 2]
            cgs[t].wait()
            if t >= 2:
                cas[t - 2].wait()

            @pl.loop(0, WIN_NODES // NGRP)
            def _grp(gi, gat=gat, abuf=abuf):
                n0 = gi * NGRP
                for c in range(HID // 16):
                    s = pl.ds(c * 16, 16)
                    for k in range(NGRP):
                        nb = (n0 + k) * 3
                        abuf[n0 + k, s] = (gat[nb, s] + gat[nb + 1, s]
                                           + gat[nb + 2, s])

            base = (w * WPW + t) * WIN_NODES
            ca = pltpu.make_async_copy(abuf, out_hbm.at[pl.ds(base, WIN_NODES)],
                                       sema)
            ca.start()
            cas[t] = ca
        cas[WPW - 2].wait()
        cas[WPW - 1].wait()

    return sc_kernel(y_pad, e_flat)


FBLK = 512


def _fin_body(a_ref, y_ref, bg_ref, wp_ref, bp_ref, o_ref, g_ref):
    i = pl.program_id(0)

    @pl.when(i == 0)
    def _():
        g_ref[...] = jnp.zeros_like(g_ref)

    row = (jax.lax.broadcasted_iota(jnp.int32, (FBLK, 1), 0) + i * FBLK)
    h = jnp.maximum(a_ref[...] + y_ref[...] + bg_ref[...], 0.0)
    h = jnp.where(row < N, h, 0.0)
    g_ref[...] += jnp.sum(h, axis=0, keepdims=True)
    o = (jnp.dot(g_ref[...], wp_ref[...], preferred_element_type=jnp.float32)
         + bp_ref[...])
    o_ref[...] = jnp.maximum(o, 0.0)


def _tc_final(a, y_pad, b_gnn, w_pool, b_pool):
    return pl.pallas_call(
        _fin_body,
        grid=(NPAD // FBLK,),
        in_specs=[
            pl.BlockSpec((FBLK, HID), lambda i: (i, 0)),
            pl.BlockSpec((FBLK, HID), lambda i: (i, 0)),
            pl.BlockSpec((1, HID), lambda i: (0, 0)),
            pl.BlockSpec((HID, HID), lambda i: (0, 0)),
            pl.BlockSpec((1, HID), lambda i: (0, 0)),
        ],
        out_specs=pl.BlockSpec((1, HID), lambda i: (0, 0)),
        out_shape=jax.ShapeDtypeStruct((1, HID), jnp.float32),
        scratch_shapes=[pltpu.VMEM((1, HID), jnp.float32)],
    )(a, y_pad, b_gnn.reshape(1, HID), w_pool, b_pool.reshape(1, HID))


def kernel(node_features, edge_index, W_gnn, b_gnn, W_pool, b_pool):
    y_pad = _tc_matmul(node_features, W_gnn)

    e_flat = jnp.pad(edge_index, ((0, NPAD - N), (0, 0)),
                     constant_values=N).reshape(NUM_WINDOWS, WIN_IDX)

    a = _sc_gather_sum(y_pad, e_flat)
    out = _tc_final(a, y_pad, b_gnn, W_pool, b_pool)
    return out.reshape(HID)


# speedup vs baseline: 1.1268x; 1.1268x over previous
import jax
import jax.numpy as jnp
from jax.experimental import pallas as pl
from jax.experimental.pallas import tpu as pltpu
from jax.experimental.pallas import tpu_sc as plsc

N = 4094
NPAD = 4096
NTIPS = 2048
HID = 128

ROW_BLK = 1024
NUM_WORKERS = 32
WIN_NODES = 32
WIN_IDX = WIN_NODES * 3
NUM_WINDOWS = NPAD // WIN_NODES
WPW = NUM_WINDOWS // NUM_WORKERS
IDX_PW = WPW * WIN_IDX
NGRP = 8


def _mm_body(x_ref, w_ref, o_ref):
    i = pl.program_id(0)
    y = jnp.dot(x_ref[...].astype(jnp.bfloat16),
                w_ref[...].astype(jnp.bfloat16),
                preferred_element_type=jnp.float32)
    row = jax.lax.broadcasted_iota(jnp.int32, (ROW_BLK, HID), 0) + i * ROW_BLK
    o_ref[...] = jnp.where(row < N, y, 0.0)


def _tc_matmul(x, w):
    return pl.pallas_call(
        _mm_body,
        grid=(NPAD // ROW_BLK,),
        in_specs=[
            pl.BlockSpec((ROW_BLK, NTIPS), lambda i: (i, 0)),
            pl.BlockSpec((NTIPS, HID), lambda i: (0, 0)),
        ],
        out_specs=pl.BlockSpec((ROW_BLK, HID), lambda i: (i, 0)),
        out_shape=jax.ShapeDtypeStruct((NPAD, HID), jnp.float32),
        compiler_params=pltpu.CompilerParams(
            dimension_semantics=("parallel",)),
    )(x, w)


def _sc_gather_pool(y_pad, e_flat, b_gnn):
    @pl.kernel(
        out_type=jax.ShapeDtypeStruct((NUM_WORKERS, HID), jnp.float32),
        mesh=plsc.VectorSubcoreMesh(core_axis_name="c", subcore_axis_name="s"),
        scratch_types=[
            pltpu.VMEM((WPW, WIN_IDX), jnp.int32),
            pltpu.VMEM((WIN_IDX, HID), jnp.float32),
            pltpu.VMEM((WIN_IDX, HID), jnp.float32),
            pltpu.VMEM((WIN_NODES, HID), jnp.float32),
            pltpu.VMEM((WIN_NODES, HID), jnp.float32),
            pltpu.VMEM((HID,), jnp.float32),
            pltpu.VMEM((HID,), jnp.float32),
            pltpu.SemaphoreType.DMA,
            pltpu.SemaphoreType.DMA,
            pltpu.SemaphoreType.DMA,
            pltpu.SemaphoreType.DMA,
            pltpu.SemaphoreType.DMA,
        ],
    )
    def sc_kernel(y_hbm, idx_hbm, b_hbm, out_hbm,
                  idx_vmem, gat0, gat1, slf0, slf1, b_vmem, acc_vmem,
                  semi, semg0, semg1, sems0, sems1):
        cid = jax.lax.axis_index("c")
        sid = jax.lax.axis_index("s")
        w = cid * 16 + sid

        pltpu.async_copy(b_hbm, b_vmem, semi).wait()
        for c in range(HID // 16):
            acc_vmem[pl.ds(c * 16, 16)] = jnp.zeros((16,), jnp.float32)

        pltpu.async_copy(idx_hbm.at[pl.ds(w * WPW, WPW)],
                         idx_vmem, semi).wait()

        bufs = ((gat0, slf0, semg0, sems0), (gat1, slf1, semg1, sems1))

        def start_win(t):
            gat, slf, semg, sems = bufs[t % 2]
            cg = pltpu.make_async_copy(y_hbm.at[idx_vmem.at[t]], gat, semg)
            cg.start()
            base = (w * WPW + t) * WIN_NODES
            cs = pltpu.make_async_copy(y_hbm.at[pl.ds(base, WIN_NODES)],
                                       slf, sems)
            cs.start()
            return cg, cs

        cps = [None] * WPW
        cps[0] = start_win(0)
        for t in range(WPW):
            if t + 1 < WPW:
                cps[t + 1] = start_win(t + 1)
            cg, cs = cps[t]
            cg.wait()
            cs.wait()
            gat, slf = bufs[t % 2][0], bufs[t % 2][1]

            @pl.loop(0, WIN_NODES // NGRP)
            def _grp(gi, gat=gat, slf=slf):
                n0 = gi * NGRP
                for c in range(HID // 16):
                    s = pl.ds(c * 16, 16)
                    b_c = b_vmem[s]
                    a_c = acc_vmem[s]
                    for k in range(NGRP):
                        nb = (n0 + k) * 3
                        v = (gat[nb, s] + gat[nb + 1, s] + gat[nb + 2, s]
                             + slf[n0 + k, s] + b_c)
                        a_c = a_c + jnp.maximum(v, 0.0)
                    acc_vmem[s] = a_c

        pltpu.async_copy(acc_vmem, out_hbm.at[w], semi).wait()

    return sc_kernel(y_pad, e_flat, b_gnn)


def _fin_body(p_ref, bg_ref, wp_ref, bp_ref, o_ref):
    g = (jnp.sum(p_ref[...], axis=0, keepdims=True)
         - 2.0 * jnp.maximum(bg_ref[...], 0.0))
    o = jnp.dot(g, wp_ref[...], preferred_element_type=jnp.float32) + bp_ref[...]
    o_ref[...] = jnp.maximum(o, 0.0)


def _tc_final(partials, b_gnn, w_pool, b_pool):
    return pl.pallas_call(
        _fin_body,
        in_specs=[
            pl.BlockSpec((NUM_WORKERS, HID), lambda: (0, 0)),
            pl.BlockSpec((1, HID), lambda: (0, 0)),
            pl.BlockSpec((HID, HID), lambda: (0, 0)),
            pl.BlockSpec((1, HID), lambda: (0, 0)),
        ],
        out_specs=pl.BlockSpec((1, HID), lambda: (0, 0)),
        out_shape=jax.ShapeDtypeStruct((1, HID), jnp.float32),
    )(partials, b_gnn.reshape(1, HID), w_pool, b_pool.reshape(1, HID))


def kernel(node_features, edge_index, W_gnn, b_gnn, W_pool, b_pool):
    y_pad = _tc_matmul(node_features, W_gnn)

    e_flat = jnp.pad(edge_index, ((0, NPAD - N), (0, 0)),
                     constant_values=N).reshape(NUM_WINDOWS, WIN_IDX)

    partials = _sc_gather_pool(y_pad, e_flat, b_gnn)
    out = _tc_final(partials, b_gnn, W_pool, b_pool)
    return out.reshape(HID)
